# baseline (speedup 1.0000x reference)
import functools

import numpy as np
import jax
import jax.numpy as jnp
from jax import lax
from jax.experimental import pallas as pl
from jax.experimental.pallas import tpu as pltpu
from jax.experimental.pallas import tpu_sc as plsc

NC = 2
NS = 16
NW = NC * NS
LANES = 16


def _pos_encoding(max_len, d):
    pos = np.arange(max_len, dtype=np.float32)[:, None]
    i = np.arange(0, d, 2, dtype=np.float32)
    angle = pos / np.power(10000.0, i / d)
    pe = np.zeros((max_len, d), dtype=np.float32)
    pe[:, 0::2] = np.sin(angle)
    pe[:, 1::2] = np.cos(angle)
    return pe


@functools.lru_cache(maxsize=None)
def _build(B, S, E, CP):
    assert S % NW == 0
    p_per_w = S // NW
    assert p_per_w % CP == 0
    n_chunks = p_per_w // CP
    nlc = E // LANES

    mesh = plsc.VectorSubcoreMesh(core_axis_name="c", subcore_axis_name="s")

    @functools.partial(
        pl.kernel,
        mesh=mesh,
        out_type=jax.ShapeDtypeStruct((B * S, E), jnp.float32),
        scratch_types=[
            pltpu.VMEM((CP,), jnp.int32),
            pltpu.VMEM((CP, E), jnp.float32),
            pltpu.VMEM((CP, E), jnp.float32),
            pltpu.SemaphoreType.DMA,
        ],
    )
    def k(x_hbm, table_hbm, pe_hbm, out_hbm, idx_v, pe_v, rows_v, sem):
        c = lax.axis_index("c")
        s = lax.axis_index("s")
        wid = s * NC + c
        p0 = wid * p_per_w

        def chunk_body(kk, _):
            pbase = p0 + kk * CP
            pltpu.sync_copy(pe_hbm.at[pl.ds(pbase, CP)], pe_v)
            for b in range(B):
                fbase = b * S + pbase
                pltpu.sync_copy(x_hbm.at[pl.ds(fbase, CP)], idx_v)
                pltpu.async_copy(table_hbm.at[idx_v], rows_v, sem).wait()

                def add_body(i, _):
                    r = i // nlc
                    col = (i % nlc) * LANES
                    rows_v[r, pl.ds(col, LANES)] = (
                        rows_v[r, pl.ds(col, LANES)]
                        + pe_v[r, pl.ds(col, LANES)]
                    )
                    return 0

                lax.fori_loop(0, CP * nlc, add_body, 0)
                pltpu.sync_copy(rows_v, out_hbm.at[pl.ds(fbase, CP)])
            return 0

        lax.fori_loop(0, n_chunks, chunk_body, 0)

    return k


def kernel(x, table):
    B, S = x.shape
    E = table.shape[1]
    pe = jnp.asarray(_pos_encoding(S, E))
    x_flat = x.reshape(B * S).astype(jnp.int32)
    out = _build(B, S, E, 32)(x_flat, table, pe)
    return out.reshape(B, S, E)


# speedup vs baseline: 2.6847x; 2.6847x over previous
import functools

import numpy as np
import jax
import jax.numpy as jnp
from jax import lax
from jax.experimental import pallas as pl
from jax.experimental.pallas import tpu as pltpu
from jax.experimental.pallas import tpu_sc as plsc

NC = 2
NS = 16
NW = NC * NS
LANES = 16


def _pos_encoding(max_len, d):
    pos = np.arange(max_len, dtype=np.float32)[:, None]
    i = np.arange(0, d, 2, dtype=np.float32)
    angle = pos / np.power(10000.0, i / d)
    pe = np.zeros((max_len, d), dtype=np.float32)
    pe[:, 0::2] = np.sin(angle)
    pe[:, 1::2] = np.cos(angle)
    return pe


@functools.lru_cache(maxsize=None)
def _build(B, S, E, CP):
    assert S % NW == 0
    p_per_w = S // NW
    assert p_per_w % CP == 0
    NG = p_per_w // CP
    assert NG % 2 == 0
    nlc = E // LANES
    row_bytes = CP * E * 4

    mesh = plsc.VectorSubcoreMesh(core_axis_name="c", subcore_axis_name="s")

    @functools.partial(
        pl.kernel,
        mesh=mesh,
        out_type=jax.ShapeDtypeStruct((B * S, E), jnp.float32),
        scratch_types=[
            pltpu.VMEM((B, p_per_w), jnp.int32),
            pltpu.VMEM((B, CP, E), jnp.float32),
            pltpu.VMEM((B, CP, E), jnp.float32),
            pltpu.VMEM((CP, E), jnp.float32),
            pltpu.VMEM((CP, E), jnp.float32),
            pltpu.SemaphoreType.DMA,
            pltpu.SemaphoreType.DMA,
            pltpu.SemaphoreType.DMA,
            pltpu.SemaphoreType.DMA,
            pltpu.SemaphoreType.DMA,
            pltpu.SemaphoreType.DMA,
        ],
    )
    def k(x_hbm, table_hbm, pe_hbm, out_hbm,
          idx_all, buf0, buf1, pe0, pe1, g0, g1, s0, s1, p0sem, p1sem):
        ci = lax.axis_index("c")
        si = lax.axis_index("s")
        wid = si * NC + ci
        p0 = wid * p_per_w

        bufs = [buf0, buf1]
        pes = [pe0, pe1]
        gsem = [g0, g1]
        ssem = [s0, s1]
        psem = [p0sem, p1sem]

        def issue_gathers(g, slot):
            for b in range(B):
                pltpu.async_copy(
                    table_hbm.at[idx_all.at[b, pl.ds(g * CP, CP)]],
                    bufs[slot].at[b], gsem[slot])

        def issue_pe(g, slot):
            pltpu.async_copy(pe_hbm.at[pl.ds(p0 + g * CP, CP)],
                             pes[slot], psem[slot])

        for b in range(B):
            pltpu.sync_copy(x_hbm.at[pl.ds(b * S + p0, p_per_w)],
                            idx_all.at[b])

        issue_gathers(0, 0)
        issue_pe(0, 0)

        def do_group(g, s):
            o = 1 - s
            pbase = p0 + g * CP
            gn = jnp.minimum(g + 1, NG - 1)
            def _wait_prev_stores():
                for b in range(B):
                    pltpu.make_async_copy(
                        bufs[o].at[b], out_hbm.at[pl.ds(b * S + pbase, CP)],
                        ssem[o]).wait()
            if s == 0:
                pl.when(g > 0)(_wait_prev_stores)
            else:
                _wait_prev_stores()
            issue_gathers(gn, o)
            issue_pe(gn, o)
            for b in range(B):
                pltpu.make_async_copy(
                    table_hbm.at[idx_all.at[b, pl.ds(g * CP, CP)]],
                    bufs[s].at[b], gsem[s]).wait()
            pltpu.make_async_copy(pe_hbm.at[pl.ds(pbase, CP)],
                                  pes[s], psem[s]).wait()

            @plsc.parallel_loop(0, CP, step=1)
            def add_body(r):
                for j in range(nlc):
                    pv = pes[s][r, pl.ds(j * LANES, LANES)]
                    for b in range(B):
                        bufs[s][b, r, pl.ds(j * LANES, LANES)] = (
                            bufs[s][b, r, pl.ds(j * LANES, LANES)] + pv)

            for b in range(B):
                pltpu.async_copy(bufs[s].at[b],
                                 out_hbm.at[pl.ds(b * S + pbase, CP)],
                                 ssem[s])

        @functools.partial(lax.fori_loop, 0, NG // 2, init_val=0)
        def _loop(gg, carry):
            do_group(2 * gg, 0)
            do_group(2 * gg + 1, 1)
            return carry

        last = p0 + (NG - 1) * CP
        for b in range(B):
            pltpu.make_async_copy(
                bufs[1].at[b], out_hbm.at[pl.ds(b * S + last, CP)],
                ssem[1]).wait()
            pltpu.make_async_copy(
                table_hbm.at[idx_all.at[b, pl.ds((NG - 1) * CP, CP)]],
                bufs[0].at[b], gsem[0]).wait()
        pltpu.make_async_copy(pe_hbm.at[pl.ds(last, CP)],
                              pes[0], psem[0]).wait()

    return k


def kernel(x, table):
    B, S = x.shape
    E = table.shape[1]
    CP = 8
    pe = jnp.asarray(_pos_encoding(S, E))
    x_flat = x.reshape(B * S).astype(jnp.int32)
    out = _build(B, S, E, CP)(x_flat, table, pe)
    return out.reshape(B, S, E)
